# baseline (speedup 1.0000x reference)
import functools

import jax
import jax.numpy as jnp
from jax import lax
from jax.experimental import pallas as pl
from jax.experimental.pallas import tpu as pltpu
from jax.experimental.pallas import tpu_sc as plsc

BATCH = 16384
EMB = 128
_HALF = BATCH // 2

_INFO = plsc.get_sparse_core_info()
_NC, _NS = _INFO.num_cores, _INFO.num_subcores
_NW = _NC * _NS
_BPW = _HALF // _NW
_CH = 128
_NCH = _BPW // _CH

_mesh = plsc.VectorSubcoreMesh(core_axis_name="c", subcore_axis_name="s")


@functools.partial(
    pl.kernel,
    mesh=_mesh,
    out_type=[
        jax.ShapeDtypeStruct((_HALF, EMB), jnp.float32),
        jax.ShapeDtypeStruct((_HALF, EMB), jnp.float32),
        jax.ShapeDtypeStruct((_HALF, 16), jnp.float32),
    ],
    scratch_types=[
        pltpu.VMEM((_NCH, _CH), jnp.int32),
        pltpu.VMEM((_NCH, _CH), jnp.int32),
        pltpu.VMEM((EMB,), jnp.float32),
        pltpu.VMEM((6, _CH, EMB), jnp.float32),
        pltpu.VMEM((_CH, 16), jnp.float32),
        [pltpu.SemaphoreType.DMA] * 6,
        [pltpu.SemaphoreType.DMA] * 2,
    ],
)
def _sc_fused(uidx_hbm, midx_hbm, ug_t, mg_t, um_t, mm_t, gmfw_hbm,
              um_o, mm_o, gmf_o,
              uvec, mvec, wvec, ring, gmfv, gsems, ssems):
    wid = lax.axis_index("s") * _NC + lax.axis_index("c")
    base = wid * _BPW
    pltpu.sync_copy(gmfw_hbm, wvec)
    for c in range(_NCH):
        pltpu.sync_copy(uidx_hbm.at[pl.ds(base + c * _CH, _CH)], uvec.at[c])
        pltpu.sync_copy(midx_hbm.at[pl.ds(base + c * _CH, _CH)], mvec.at[c])

    def g(tab, ivec, c, s):
        return pltpu.make_async_copy(tab.at[ivec.at[c]], ring.at[s], gsems[s])

    def st(out, c, s, ss):
        return pltpu.make_async_copy(ring.at[s],
                                     out.at[pl.ds(base + c * _CH, _CH)],
                                     ssems[ss])

    def dot_chunk(su, sv, c):
        ugb = ring.at[su]
        mgb = ring.at[sv]

        def group(gi, _):
            for r in range(16):
                row = gi * 16 + r
                acc = (ugb[row, pl.ds(0, 16)] * mgb[row, pl.ds(0, 16)]
                       * wvec[pl.ds(0, 16)])
                for k in range(1, EMB // 16):
                    acc = acc + (ugb[row, pl.ds(16 * k, 16)]
                                 * mgb[row, pl.ds(16 * k, 16)]
                                 * wvec[pl.ds(16 * k, 16)])
                gmfv[gi * 16 + r] = acc
            return 0

        lax.fori_loop(0, _CH // 16, group, 0)

    g(um_t, uvec, 0, 0).start()
    g(mm_t, mvec, 0, 1).start()
    g(ug_t, uvec, 0, 2).start()
    g(mg_t, mvec, 0, 3).start()
    g(ug_t, uvec, 1, 4).start()
    g(mg_t, mvec, 1, 5).start()
    g(um_t, uvec, 0, 0).wait()
    st(um_o, 0, 0, 0).start()
    g(mm_t, mvec, 0, 1).wait()
    st(mm_o, 0, 1, 1).start()
    g(ug_t, uvec, 0, 2).wait()
    g(mg_t, mvec, 0, 3).wait()
    dot_chunk(2, 3, 0)
    pltpu.sync_copy(gmfv, gmf_o.at[pl.ds(base, _CH)])
    st(um_o, 0, 0, 0).wait()
    g(um_t, uvec, 1, 0).start()
    st(mm_o, 0, 1, 1).wait()
    g(mm_t, mvec, 1, 1).start()
    g(um_t, uvec, 1, 0).wait()
    st(um_o, 1, 0, 0).start()
    g(mm_t, mvec, 1, 1).wait()
    st(mm_o, 1, 1, 1).start()
    g(ug_t, uvec, 1, 4).wait()
    g(mg_t, mvec, 1, 5).wait()
    dot_chunk(4, 5, 1)
    pltpu.sync_copy(gmfv, gmf_o.at[pl.ds(base + _CH, _CH)])
    st(um_o, 1, 0, 0).wait()
    st(mm_o, 1, 1, 1).wait()


_BM = 2048


def _nt(a, b):
    return lax.dot_general(a, b, (((1,), (1,)), ((), ())),
                           preferred_element_type=jnp.float32)


def _tc_body(um, mm, gp, w0at, w0bt, b0c, w1t, b1c, w2t, b2c,
             w3t, b3c, fmwt, cconst, out_ref):
    h = jnp.maximum(_nt(w0at[...], um[...]) + _nt(w0bt[...], mm[...])
                    + b0c[...], 0.0)
    h = jnp.maximum(jnp.dot(w1t[...], h, preferred_element_type=jnp.float32)
                    + b1c[...], 0.0)
    h = jnp.maximum(jnp.dot(w2t[...], h, preferred_element_type=jnp.float32)
                    + b2c[...], 0.0)
    h = jnp.maximum(jnp.dot(w3t[...], h, preferred_element_type=jnp.float32)
                    + b3c[...], 0.0)
    m = jnp.dot(fmwt[...], h, preferred_element_type=jnp.float32)
    g = _nt(jnp.ones((1, 16), jnp.float32), gp[...])
    out_ref[...] = (m + g + cconst[...])[0]


def _full(shape):
    return pl.BlockSpec(shape, lambda i: (0, 0))


def _row(shape):
    return pl.BlockSpec(shape, lambda i: (i, 0))


_tc_call = pl.pallas_call(
    _tc_body,
    grid=(_HALF // _BM,),
    in_specs=[
        _row((_BM, EMB)),
        _row((_BM, EMB)),
        _row((_BM, 16)),
        _full((64, EMB)),
        _full((64, EMB)),
        _full((64, 1)),
        _full((32, 64)),
        _full((32, 1)),
        _full((16, 32)),
        _full((16, 1)),
        _full((8, 16)),
        _full((8, 1)),
        _full((1, 8)),
        _full((1, 1)),
    ],
    out_specs=pl.BlockSpec((_BM,), lambda i: (i,)),
    out_shape=jax.ShapeDtypeStruct((_HALF,), jnp.float32),
)


def kernel(X, user_emb_gmf, movie_emb_gmf, user_emb_mlp, movie_emb_mlp,
           gmf_w, gmf_b, final_mlp_w, final_mlp_b, final_w, final_b,
           mlp_w0, mlp_b0, mlp_w1, mlp_b1, mlp_w2, mlp_b2, mlp_w3, mlp_b3):
    user = X[:, 0]
    movie = X[:, 1]
    fw0 = final_w[0, 0]
    fw1 = final_w[1, 0]
    gmfw = gmf_w[:, 0] * fw0
    fmwt = (final_mlp_w[:, 0] * fw1).reshape(1, 8)
    cconst = (final_b[0] + fw0 * gmf_b[0] + fw1 * final_mlp_b[0]).reshape(1, 1)
    wts = (mlp_w0[:EMB].T, mlp_w0[EMB:].T, mlp_b0.reshape(-1, 1),
           mlp_w1.T, mlp_b1.reshape(-1, 1), mlp_w2.T, mlp_b2.reshape(-1, 1),
           mlp_w3.T, mlp_b3.reshape(-1, 1), fmwt, cconst)
    outs = []
    for h in range(2):
        sl = slice(h * _HALF, (h + 1) * _HALF)
        um, mm, gp = _sc_fused(user[sl], movie[sl],
                               user_emb_gmf, movie_emb_gmf,
                               user_emb_mlp, movie_emb_mlp, gmfw)
        outs.append(_tc_call(um, mm, gp, *wts))
    return jnp.concatenate(outs).reshape(BATCH, 1)
